# baseline (speedup 1.0000x reference)
import functools
import math

import jax
import jax.numpy as jnp
import numpy as np
from jax import lax
from jax.experimental import pallas as pl
from jax.experimental.pallas import tpu as pltpu
from jax.experimental.pallas import tpu_sc as plsc

NW = 32

_CACHE = {}


def _round8(x):
    return ((x + 7) // 8) * 8


def _tiles(total, t):
    out = [(s, t) for s in range(0, total - t + 1, t)]
    rem = total % t
    if rem:
        out.append((total - rem, rem))
    return out


def _indices(L, n0, n1, C0, C1, full0, full1):
    key = ("idx", L, n0)
    if key in _CACHE:
        return _CACHE[key]
    try:
        with jax.ensure_compile_time_eval(), \
                jax.default_device(jax.local_devices(backend="cpu")[0]):
            perm = np.asarray(
                jax.random.permutation(jax.random.key(42), L)
            ).astype(np.int32)
    except Exception:
        perm = jax.random.permutation(jax.random.key(42), L).astype(
            jnp.int32)
    idx0, idx1 = perm[:n0], perm[n0:]
    xp = np if isinstance(perm, np.ndarray) else jnp

    def chunk(idx, j, C, full):
        n = idx.shape[0]
        if j < full:
            return idx[j * C:(j + 1) * C]
        if j == full and n % C:
            rem = n - full * C
            a8 = rem - (rem % 8)
            pieces = [idx[full * C:full * C + a8]]
            used = a8
            if rem % 8:
                pieces.append(idx[n - 16:n])
                used += 16
            pieces.append(xp.zeros(C - used, xp.int32))
            return xp.concatenate(pieces)
        return xp.zeros(C, xp.int32)

    ia0 = xp.concatenate([chunk(idx0, w, C0, full0) for w in range(NW)])
    ia1 = xp.concatenate(
        [chunk(idx1, NW - 1 - w, C1, full1) for w in range(NW)])
    ip0 = xp.concatenate([idx0, xp.zeros(NW * C0 - n0, xp.int32)])
    ip1 = xp.concatenate([idx1, xp.zeros(NW * C1 - n1, xp.int32)])
    out = (ia0, ia1, ip0, ip1)
    if xp is np:
        _CACHE[key] = out
    return out


def _build(L, D, n0, n1, C0, C1, full0, full1):
    key = ("k", L, D, n0)
    if key in _CACHE:
        return _CACHE[key]
    T = 16
    tail0 = n0 - full0 * C0
    tail1 = n1 - full1 * C1
    mesh = plsc.VectorSubcoreMesh(core_axis_name="c", subcore_axis_name="s")

    def role0(w):
        if w < full0:
            return "full"
        return "tail" if (w == full0 and tail0) else "none"

    def role1(w):
        j = NW - 1 - w
        if j < full1:
            return "full"
        return "tail" if (j == full1 and tail1) else "none"

    groups = {}
    for w in range(NW):
        groups.setdefault((role0(w), role1(w)), []).append(w)
    for ws in groups.values():
        assert ws == list(range(ws[0], ws[-1] + 1))

    scratch = [
        pltpu.VMEM((C0,), jnp.int32),
        pltpu.VMEM((C1,), jnp.int32),
        pltpu.VMEM((C0,), jnp.int32),
        pltpu.VMEM((C1,), jnp.int32),
        pltpu.VMEM((T, D), jnp.float32),
        pltpu.VMEM((T, D), jnp.float32),
        pltpu.VMEM((64, 128), jnp.float32),
        pltpu.VMEM((64, 128), jnp.float32),
    ] + [pltpu.SemaphoreType.DMA] * 8

    @functools.partial(
        pl.kernel,
        mesh=mesh,
        out_type=(
            jax.ShapeDtypeStruct((n0, D), jnp.float32),
            jax.ShapeDtypeStruct((n1, D), jnp.float32),
            jax.ShapeDtypeStruct((NW * C0, 128), jnp.float32),
            jax.ShapeDtypeStruct((NW * C1, 128), jnp.float32),
        ),
        scratch_types=scratch,
        compiler_params=pltpu.CompilerParams(use_tc_tiling_on_sc=False),
    )
    def gather_kernel(e0, e1, c0, c1, ia0, ia1, ip0, ip1,
                      o0, o1, oc0, oc1,
                      ia0v, ia1v, ip0v, ip1v, bufA, bufB, cbA, cbB,
                      rsA, rsB, wsA, wsB, crsA, crsB, cwsA, cwsB):
        wid = lax.axis_index("s") * 2 + lax.axis_index("c")
        ih = [
            pltpu.async_copy(ia0.at[pl.ds(wid * C0, C0)], ia0v, crsA),
            pltpu.async_copy(ia1.at[pl.ds(wid * C1, C1)], ia1v, crsA),
            pltpu.async_copy(ip0.at[pl.ds(wid * C0, C0)], ip0v, crsA),
            pltpu.async_copy(ip1.at[pl.ds(wid * C1, C1)], ip1v, crsA),
        ]
        for h in ih:
            h.wait()

        BUF = ((bufA, bufB), (cbA, cbB))
        RS = ((rsA, rsB), (crsA, crsB))
        WS = ((wsA, wsB), (cwsA, cwsB))

        def pipe(items):
            n = len(items)
            occ, cnt = [], {0: 0, 1: 0}
            for s, *_ in items:
                occ.append(cnt[s])
                cnt[s] += 1
            prev = {}
            for k, (s, *_) in enumerate(items):
                prev[(s, occ[k])] = k
            rh, wh, waited = {}, {}, set()

            def gstart(k):
                s, src, iv, r, t, _, _ = items[k]
                b = occ[k] % 2
                if occ[k] >= 2:
                    j = prev[(s, occ[k] - 2)]
                    if j not in waited:
                        wh[j].wait()
                        waited.add(j)
                rh[k] = pltpu.async_copy(
                    src.at[iv.at[pl.ds(r, t)]],
                    BUF[s][b].at[pl.ds(0, t)], RS[s][b])

            gstart(0)
            for k in range(n):
                s, _, _, r, t, out, ob = items[k]
                b = occ[k] % 2
                if k + 1 < n:
                    gstart(k + 1)
                rh[k].wait()
                wh[k] = pltpu.async_copy(
                    BUF[s][b].at[pl.ds(0, t)],
                    out.at[pl.ds(ob + r, t)], WS[s][b])
            for k in range(n):
                if k not in waited:
                    wh[k].wait()

        def span_items(src, iv, out, ob, total):
            a8 = total - (total % 8)
            return [(0, src, iv, r, t, out, ob) for r, t in _tiles(a8, T)]

        def tail_reg(src, iv, out, ob, total):
            a8 = total - (total % 8)
            ids = lax.iota(jnp.int32, 16) + (ob + total - 16)
            gidx = iv[pl.ds(a8, 16)]
            pltpu.async_copy(src.at[gidx], bufA.at[pl.ds(0, 16)], rsA).wait()
            pltpu.async_copy(bufA.at[pl.ds(0, 16)], out.at[ids], wsA).wait()

        coord_items = (
            [(1, c0, ip0v, r, t, oc0, wid * C0) for r, t in _tiles(C0, 64)]
            + [(1, c1, ip1v, r, t, oc1, wid * C1) for r, t in _tiles(C1, 64)])

        for (r0, r1), wlist in groups.items():
            lo, hi = wlist[0], wlist[-1]
            pred = (wid >= lo) & (wid <= hi)

            @pl.when(pred)
            def _(r0=r0, r1=r1):
                items = []
                if r0 == "full":
                    items += span_items(e0, ia0v, o0, wid * C0, C0)
                elif r0 == "tail":
                    items += span_items(e0, ia0v, o0, full0 * C0, tail0)
                if r1 == "full":
                    items += span_items(e1, ia1v, o1, (NW - 1 - wid) * C1, C1)
                elif r1 == "tail":
                    items += span_items(e1, ia1v, o1, full1 * C1, tail1)
                items = items[:1] + coord_items + items[1:]
                pipe(items)
                if r0 == "tail" and tail0 % 8:
                    tail_reg(e0, ia0v, o0, full0 * C0, tail0)
                if r1 == "tail" and tail1 % 8:
                    tail_reg(e1, ia1v, o1, full1 * C1, tail1)

    _CACHE[key] = gather_kernel
    return gather_kernel


def kernel(embeddings_0, embeddings_1, coords_0, coords_1):
    L, D = embeddings_0.shape
    n0 = int(0.7 * L)
    n1 = L - n0
    C0 = _round8(math.ceil(n0 / NW))
    C1 = _round8(math.ceil(n1 / NW))
    full0, full1 = n0 // C0, n1 // C1
    ia0, ia1, ip0, ip1 = _indices(L, n0, n1, C0, C1, full0, full1)
    f = _build(L, D, n0, n1, C0, C1, full0, full1)
    c0p = jnp.pad(coords_0, ((0, 0), (0, 125)))
    c1p = jnp.pad(coords_1, ((0, 0), (0, 125)))
    o0, o1, oc0, oc1 = f(
        embeddings_0, embeddings_1, c0p, c1p, ia0, ia1, ip0, ip1,
    )
    return o0, o1, oc0[:n0, :3], oc1[:n1, :3]


# speedup vs baseline: 2.5622x; 2.5622x over previous
import functools
import math

import jax
import jax.numpy as jnp
import numpy as np
from jax import lax
from jax.experimental import pallas as pl
from jax.experimental.pallas import tpu as pltpu
from jax.experimental.pallas import tpu_sc as plsc

NW = 32

_CACHE = {}


def _round8(x):
    return ((x + 7) // 8) * 8


def _tiles(total, t):
    out = [(s, t) for s in range(0, total - t + 1, t)]
    rem = total % t
    if rem:
        out.append((total - rem, rem))
    return out


def _indices(L, n0, n1, C0, C1, full0, full1):
    key = ("idx", L, n0)
    if key in _CACHE:
        return _CACHE[key]
    try:
        with jax.ensure_compile_time_eval(), \
                jax.default_device(jax.local_devices(backend="cpu")[0]):
            perm = np.asarray(
                jax.random.permutation(jax.random.key(42), L)
            ).astype(np.int32)
    except Exception:
        perm = jax.random.permutation(jax.random.key(42), L).astype(
            jnp.int32)
    idx0, idx1 = perm[:n0], perm[n0:]
    xp = np if isinstance(perm, np.ndarray) else jnp

    def chunk(idx, j, C, full):
        n = idx.shape[0]
        if j < full:
            return idx[j * C:(j + 1) * C]
        if j == full and n % C:
            rem = n - full * C
            a8 = rem - (rem % 8)
            pieces = [idx[full * C:full * C + a8]]
            used = a8
            if rem % 8:
                pieces.append(idx[n - 16:n])
                used += 16
            pieces.append(xp.zeros(C - used, xp.int32))
            return xp.concatenate(pieces)
        return xp.zeros(C, xp.int32)

    ia0 = xp.concatenate([chunk(idx0, w, C0, full0) for w in range(NW)])
    ia1 = xp.concatenate(
        [chunk(idx1, NW - 1 - w, C1, full1) for w in range(NW)])
    ip0 = xp.concatenate([idx0, xp.zeros(NW * C0 - n0, xp.int32)])
    ip1 = xp.concatenate([idx1, xp.zeros(NW * C1 - n1, xp.int32)])
    out = (ia0, ia1, ip0, ip1)
    if xp is np:
        _CACHE[key] = out
    return out


def _build(L, D, n0, n1, C0, C1, full0, full1):
    key = ("k", L, D, n0)
    if key in _CACHE:
        return _CACHE[key]
    T = 16
    tail0 = n0 - full0 * C0
    tail1 = n1 - full1 * C1
    mesh = plsc.VectorSubcoreMesh(core_axis_name="c", subcore_axis_name="s")

    def role0(w):
        if w < full0:
            return "full"
        return "tail" if (w == full0 and tail0) else "none"

    def role1(w):
        j = NW - 1 - w
        if j < full1:
            return "full"
        return "tail" if (j == full1 and tail1) else "none"

    groups = {}
    for w in range(NW):
        groups.setdefault((role0(w), role1(w)), []).append(w)
    for ws in groups.values():
        assert ws == list(range(ws[0], ws[-1] + 1))

    scratch = [
        pltpu.VMEM((C0,), jnp.int32),
        pltpu.VMEM((C1,), jnp.int32),
        pltpu.VMEM((C0,), jnp.int32),
        pltpu.VMEM((C1,), jnp.int32),
        pltpu.VMEM((T, D), jnp.float32),
        pltpu.VMEM((T, D), jnp.float32),
        pltpu.VMEM((64, 128), jnp.float32),
        pltpu.VMEM((64, 128), jnp.float32),
    ] + [pltpu.SemaphoreType.DMA] * 8

    @functools.partial(
        pl.kernel,
        mesh=mesh,
        out_type=(
            jax.ShapeDtypeStruct((n0, D), jnp.float32),
            jax.ShapeDtypeStruct((n1, D), jnp.float32),
            jax.ShapeDtypeStruct((NW * C0, 128), jnp.float32),
            jax.ShapeDtypeStruct((NW * C1, 128), jnp.float32),
        ),
        scratch_types=scratch,
    )
    def gather_kernel(e0, e1, c0, c1, ia0, ia1, ip0, ip1,
                      o0, o1, oc0, oc1,
                      ia0v, ia1v, ip0v, ip1v, bufA, bufB, cbA, cbB,
                      rsA, rsB, wsA, wsB, crsA, crsB, cwsA, cwsB):
        wid = lax.axis_index("c") * 16 + lax.axis_index("s")
        ih = [
            pltpu.async_copy(ia0.at[pl.ds(wid * C0, C0)], ia0v, crsA),
            pltpu.async_copy(ia1.at[pl.ds(wid * C1, C1)], ia1v, crsA),
            pltpu.async_copy(ip0.at[pl.ds(wid * C0, C0)], ip0v, crsA),
            pltpu.async_copy(ip1.at[pl.ds(wid * C1, C1)], ip1v, crsA),
        ]
        for h in ih:
            h.wait()

        BUF = ((bufA, bufB), (cbA, cbB))
        RS = ((rsA, rsB), (crsA, crsB))
        WS = ((wsA, wsB), (cwsA, cwsB))

        def pipe(items):
            n = len(items)
            occ, cnt = [], {0: 0, 1: 0}
            for s, *_ in items:
                occ.append(cnt[s])
                cnt[s] += 1
            prev = {}
            for k, (s, *_) in enumerate(items):
                prev[(s, occ[k])] = k
            rh, wh, waited = {}, {}, set()

            def gstart(k):
                s, src, iv, r, t, _, _ = items[k]
                b = occ[k] % 2
                if occ[k] >= 2:
                    j = prev[(s, occ[k] - 2)]
                    if j not in waited:
                        wh[j].wait()
                        waited.add(j)
                rh[k] = pltpu.async_copy(
                    src.at[iv.at[pl.ds(r, t)]],
                    BUF[s][b].at[pl.ds(0, t)], RS[s][b])

            gstart(0)
            for k in range(n):
                s, _, _, r, t, out, ob = items[k]
                b = occ[k] % 2
                if k + 1 < n:
                    gstart(k + 1)
                rh[k].wait()
                wh[k] = pltpu.async_copy(
                    BUF[s][b].at[pl.ds(0, t)],
                    out.at[pl.ds(ob + r, t)], WS[s][b])
            for k in range(n):
                if k not in waited:
                    wh[k].wait()

        def span_items(src, iv, out, ob, total):
            a8 = total - (total % 8)
            return [(0, src, iv, r, t, out, ob) for r, t in _tiles(a8, T)]

        def tail_reg(src, iv, out, ob, total):
            a8 = total - (total % 8)
            ids = lax.iota(jnp.int32, 16) + (ob + total - 16)
            gidx = iv[pl.ds(a8, 16)]
            pltpu.async_copy(src.at[gidx], bufA.at[pl.ds(0, 16)], rsA).wait()
            pltpu.async_copy(bufA.at[pl.ds(0, 16)], out.at[ids], wsA).wait()

        coord_items = (
            [(1, c0, ip0v, r, t, oc0, wid * C0) for r, t in _tiles(C0, 64)]
            + [(1, c1, ip1v, r, t, oc1, wid * C1) for r, t in _tiles(C1, 64)])

        for (r0, r1), wlist in groups.items():
            lo, hi = wlist[0], wlist[-1]
            pred = (wid >= lo) & (wid <= hi)

            @pl.when(pred)
            def _(r0=r0, r1=r1):
                items = []
                if r0 == "full":
                    items += span_items(e0, ia0v, o0, wid * C0, C0)
                elif r0 == "tail":
                    items += span_items(e0, ia0v, o0, full0 * C0, tail0)
                if r1 == "full":
                    items += span_items(e1, ia1v, o1, (NW - 1 - wid) * C1, C1)
                elif r1 == "tail":
                    items += span_items(e1, ia1v, o1, full1 * C1, tail1)
                items = items[:1] + coord_items + items[1:]
                pipe(items)
                if r0 == "tail" and tail0 % 8:
                    tail_reg(e0, ia0v, o0, full0 * C0, tail0)
                if r1 == "tail" and tail1 % 8:
                    tail_reg(e1, ia1v, o1, full1 * C1, tail1)

    _CACHE[key] = gather_kernel
    return gather_kernel


def kernel(embeddings_0, embeddings_1, coords_0, coords_1):
    L, D = embeddings_0.shape
    n0 = int(0.7 * L)
    n1 = L - n0
    C0 = _round8(math.ceil(n0 / NW))
    C1 = _round8(math.ceil(n1 / NW))
    full0, full1 = n0 // C0, n1 // C1
    ia0, ia1, ip0, ip1 = _indices(L, n0, n1, C0, C1, full0, full1)
    f = _build(L, D, n0, n1, C0, C1, full0, full1)
    c0p = jnp.pad(coords_0, ((0, 0), (0, 125)))
    c1p = jnp.pad(coords_1, ((0, 0), (0, 125)))
    o0, o1, oc0, oc1 = f(
        embeddings_0, embeddings_1, c0p, c1p, ia0, ia1, ip0, ip1,
    )
    return o0, o1, oc0[:n0, :3], oc1[:n1, :3]
